# baseline (speedup 1.0000x reference)
import functools

import jax
import jax.numpy as jnp
from jax import lax
from jax.experimental import pallas as pl
from jax.experimental.pallas import tpu as pltpu
from jax.experimental.pallas import tpu_sc as plsc

N = 10000
E = 320000
G = 64
NC, NS = 2, 16
NW = NC * NS
CH = 128
EPAD = 327680
NROWS = EPAD // CH
RPW = NROWS // NW
BL = 16
NBLK = RPW // BL
ACCN = N + 8
RPT = 624
RPT_LAST = N - (NS - 1) * RPT
RPTZ_LAST = ACCN - (NS - 1) * RPT


def _sc_scatter(F):
    mesh = plsc.VectorSubcoreMesh(core_axis_name="c", subcore_axis_name="s",
                                  num_cores=NC, num_subcores=NS)

    @functools.partial(
        pl.kernel,
        out_type=jax.ShapeDtypeStruct((NC, N, F), jnp.float32),
        mesh=mesh,
        scratch_types=[
            pltpu.VMEM((CH,), jnp.int32),
            pltpu.VMEM((CH,), jnp.int32),
            pltpu.VMEM((CH, F), jnp.float32),
            pltpu.VMEM_SHARED((ACCN, F), jnp.float32),
            pltpu.SemaphoreType.DMA,
        ],
        compiler_params=pltpu.CompilerParams(use_tc_tiling_on_sc=False),
    )
    def scatter_kernel(h_hbm, src_hbm, dst_hbm, zeros_hbm, out_hbm,
                       sidx, didx, rows, acc, sem):
        c = lax.axis_index("c")
        s = lax.axis_index("s")
        start = s * RPT

        @pl.when(s < NS - 1)
        def _():
            pltpu.sync_copy(zeros_hbm.at[pl.ds(0, RPT)],
                            acc.at[pl.ds(start, RPT)])

        @pl.when(s == NS - 1)
        def _():
            pltpu.sync_copy(zeros_hbm, acc.at[pl.ds(start, RPTZ_LAST)])

        plsc.subcore_barrier()
        rowbase = (c * NS + s) * RPW

        def body(i, carry):
            r = rowbase + i
            pltpu.sync_copy(src_hbm.at[r], sidx)
            pltpu.sync_copy(dst_hbm.at[r], didx)
            pltpu.async_copy(h_hbm.at[sidx], rows, sem).wait()
            pltpu.sync_copy(rows, acc.at[didx], add=True)
            return carry

        lax.fori_loop(0, RPW, body, 0)
        plsc.subcore_barrier()

        @pl.when(s < NS - 1)
        def _():
            pltpu.sync_copy(acc.at[pl.ds(start, RPT)],
                            out_hbm.at[c, pl.ds(start, RPT)])

        @pl.when(s == NS - 1)
        def _():
            pltpu.sync_copy(acc.at[pl.ds(start, RPT_LAST)],
                            out_hbm.at[c, pl.ds(start, RPT_LAST)])

    return scatter_kernel


_SCAT64 = _sc_scatter(64)
_SCAT128 = _sc_scatter(128)


def _leaky(x):
    return jnp.where(x >= 0, x, 0.01 * x)


_BM = 1000
_BC = 2000


def _mm1_body(x_ref, w_ref, *y_refs):
    x = x_ref[...]
    for k in range(4):
        y_refs[k][...] = jnp.dot(x, w_ref[k],
                                 preferred_element_type=jnp.float32)


_MM1 = pl.pallas_call(
    _mm1_body,
    grid=(N // _BM,),
    in_specs=[pl.BlockSpec((_BM, 128), lambda i: (i, 0)),
              pl.BlockSpec((4, 128, 64), lambda i: (0, 0, 0))],
    out_specs=[pl.BlockSpec((_BM, 64), lambda i: (i, 0))] * 4,
    out_shape=[jax.ShapeDtypeStruct((N, 64), jnp.float32)] * 4,
)


def _make_combine(F, n_extra, bias, act):
    def body(*refs):
        refs = list(refs)
        o_ref = refs.pop()
        b_ref = refs.pop() if bias else None
        p_ref = refs.pop(0)
        t = p_ref[0] + p_ref[1]
        for r in refs:
            t = t + r[...]
        if b_ref is not None:
            t = t + b_ref[...]
        if act:
            t = _leaky(t)
        o_ref[...] = t

    in_specs = [pl.BlockSpec((2, _BC, F), lambda i: (0, i, 0))]
    in_specs += [pl.BlockSpec((_BC, F), lambda i: (i, 0))] * n_extra
    if bias:
        in_specs.append(pl.BlockSpec((1, F), lambda i: (0, 0)))
    return pl.pallas_call(
        body,
        grid=(N // _BC,),
        in_specs=in_specs,
        out_specs=pl.BlockSpec((_BC, F), lambda i: (i, 0)),
        out_shape=jax.ShapeDtypeStruct((N, F), jnp.float32),
    )


_COMB64_Y = _make_combine(64, 1, False, False)
_COMB64_YBA = _make_combine(64, 1, True, True)
_COMB64 = _make_combine(64, 0, False, False)
_COMB128 = _make_combine(128, 0, False, False)


def _make_mm4(fin, fout, act):
    def body(h0, h1, h2, h3, w_ref, b_ref, o_ref):
        acc = jnp.dot(h0[...], w_ref[0], preferred_element_type=jnp.float32)
        for k, h in enumerate((h1, h2, h3), start=1):
            acc = acc + jnp.dot(h[...], w_ref[k],
                                preferred_element_type=jnp.float32)
        acc = acc + b_ref[...]
        if act:
            acc = _leaky(acc)
        o_ref[...] = acc

    return pl.pallas_call(
        body,
        grid=(N // _BM,),
        in_specs=[pl.BlockSpec((_BM, fin), lambda i: (i, 0))] * 4
        + [pl.BlockSpec((4, fin, fout), lambda i: (0, 0, 0)),
           pl.BlockSpec((1, fout), lambda i: (0, 0))],
        out_specs=pl.BlockSpec((_BM, fout), lambda i: (i, 0)),
        out_shape=jax.ShapeDtypeStruct((N, fout), jnp.float32),
    )


_MM4_L2 = _make_mm4(64, 128, True)
_MM4_L3 = _make_mm4(128, 256, False)

_BP = 1000


def _pool_body(b_ref, h_ref, o_ref):
    @pl.when(pl.program_id(0) == 0)
    def _init():
        o_ref[...] = jnp.full((G, 256), -jnp.inf, jnp.float32)

    bi = b_ref[0]
    h = h_ref[...]
    lo = jnp.min(bi)
    hi = jnp.max(bi)

    def gbody(g, carry):
        m = bi == g
        v = jnp.max(jnp.where(m, h, -jnp.inf), axis=0, keepdims=True)
        o_ref[pl.ds(g, 1), :] = jnp.maximum(o_ref[pl.ds(g, 1), :], v)
        return carry

    lax.fori_loop(lo, hi + 1, gbody, 0)


_POOL = pl.pallas_call(
    _pool_body,
    grid=(N // _BP,),
    in_specs=[pl.BlockSpec((1, _BP, 1), lambda i: (i, 0, 0)),
              pl.BlockSpec((_BP, 256), lambda i: (i, 0))],
    out_specs=pl.BlockSpec((G, 256), lambda i: (0, 0)),
    out_shape=jax.ShapeDtypeStruct((G, 256), jnp.float32),
)


def _mlp_body(g_ref, w1, b1, w2, b2, w3, b3, o_ref):
    g = g_ref[...]
    g = jnp.where(jnp.isfinite(g), g, 0.0)
    h = jnp.maximum(
        jnp.dot(g, w1[...], preferred_element_type=jnp.float32) + b1[...], 0.0)
    h = jnp.maximum(
        jnp.dot(h, w2[...], preferred_element_type=jnp.float32) + b2[...], 0.0)
    o_ref[...] = jnp.dot(h, w3[...], preferred_element_type=jnp.float32) + b3[...]


_MLP = pl.pallas_call(
    _mlp_body,
    out_shape=jax.ShapeDtypeStruct((G, 4), jnp.float32),
)


def kernel(x, edge_index, batch, W1, b1, W2, b2, W3, b3,
           Wc1, bc1, Wc2, bc2, Wc3, bc3):
    pad = EPAD - E
    src = jnp.concatenate(
        [edge_index[0], jnp.zeros((pad,), edge_index.dtype)]).reshape(NROWS, CH)
    dst = jnp.concatenate(
        [edge_index[1], jnp.full((pad,), N, edge_index.dtype)]).reshape(NROWS, CH)
    z64 = jnp.zeros((RPTZ_LAST, 64), jnp.float32)
    z128 = jnp.zeros((RPTZ_LAST, 128), jnp.float32)

    y0, y1, y2, y3 = _MM1(x, W1)
    p = _SCAT64(y3, src, dst, z64)
    t = _COMB64_Y(p, y2)
    p = _SCAT64(t, src, dst, z64)
    t = _COMB64_Y(p, y1)
    p = _SCAT64(t, src, dst, z64)
    h1 = _COMB64_YBA(p, y0, b1.reshape(1, 64))

    p = _SCAT64(h1, src, dst, z64)
    h2 = _COMB64(p)
    p = _SCAT64(h2, src, dst, z64)
    h3 = _COMB64(p)
    p = _SCAT64(h3, src, dst, z64)
    h4 = _COMB64(p)
    z = _MM4_L2(h1, h2, h3, h4, W2, b2.reshape(1, 128))

    p = _SCAT128(z, src, dst, z128)
    g2 = _COMB128(p)
    p = _SCAT128(g2, src, dst, z128)
    g3 = _COMB128(p)
    p = _SCAT128(g3, src, dst, z128)
    g4 = _COMB128(p)
    hfin = _MM4_L3(z, g2, g3, g4, W3, b3.reshape(1, 256))

    gpool = _POOL(batch.reshape(N // _BP, _BP, 1), hfin)
    return _MLP(gpool, Wc1, bc1.reshape(1, 1024), Wc2, bc2.reshape(1, 512),
                Wc3, bc3.reshape(1, 4))


# speedup vs baseline: 2.0016x; 2.0016x over previous
import functools

import jax
import jax.numpy as jnp
from jax import lax
from jax.experimental import pallas as pl
from jax.experimental.pallas import tpu as pltpu
from jax.experimental.pallas import tpu_sc as plsc

N = 10000
E = 320000
G = 64
NC, NS = 2, 16
NW = NC * NS
EPW = E // NW
CH = 128
NFULL = EPW // CH
TAIL = EPW - NFULL * CH
RPT = 624
RPT_LAST = N - (NS - 1) * RPT


def _sc_scatter(F):
    mesh = plsc.VectorSubcoreMesh(core_axis_name="c", subcore_axis_name="s",
                                  num_cores=NC, num_subcores=NS)

    @functools.partial(
        pl.kernel,
        out_type=jax.ShapeDtypeStruct((NC, N, F), jnp.float32),
        mesh=mesh,
        scratch_types=[
            pltpu.VMEM((CH,), jnp.int32),
            pltpu.VMEM((CH,), jnp.int32),
            pltpu.VMEM((CH, F), jnp.float32),
            pltpu.VMEM((TAIL,), jnp.int32),
            pltpu.VMEM((TAIL,), jnp.int32),
            pltpu.VMEM((TAIL, F), jnp.float32),
            pltpu.VMEM_SHARED((N, F), jnp.float32),
            pltpu.SemaphoreType.DMA,
        ],
        compiler_params=pltpu.CompilerParams(use_tc_tiling_on_sc=False),
    )
    def scatter_kernel(h_hbm, src_hbm, dst_hbm, zeros_hbm, out_hbm,
                       sidx, didx, rows, sidx_t, didx_t, rows_t, acc, sem):
        c = lax.axis_index("c")
        s = lax.axis_index("s")
        start = s * RPT

        @pl.when(s < NS - 1)
        def _():
            pltpu.sync_copy(zeros_hbm.at[pl.ds(0, RPT)],
                            acc.at[pl.ds(start, RPT)])

        @pl.when(s == NS - 1)
        def _():
            pltpu.sync_copy(zeros_hbm, acc.at[pl.ds(start, RPT_LAST)])

        plsc.subcore_barrier()
        base = (c * NS + s) * EPW

        def body(i, carry):
            off = base + i * CH
            pltpu.sync_copy(src_hbm.at[pl.ds(off, CH)], sidx)
            pltpu.sync_copy(dst_hbm.at[pl.ds(off, CH)], didx)
            pltpu.async_copy(h_hbm.at[sidx], rows, sem).wait()
            pltpu.sync_copy(rows, acc.at[didx], add=True)
            return carry

        lax.fori_loop(0, NFULL, body, 0)
        toff = base + NFULL * CH
        pltpu.sync_copy(src_hbm.at[pl.ds(toff, TAIL)], sidx_t)
        pltpu.sync_copy(dst_hbm.at[pl.ds(toff, TAIL)], didx_t)
        pltpu.async_copy(h_hbm.at[sidx_t], rows_t, sem).wait()
        pltpu.sync_copy(rows_t, acc.at[didx_t], add=True)
        plsc.subcore_barrier()

        @pl.when(s < NS - 1)
        def _():
            pltpu.sync_copy(acc.at[pl.ds(start, RPT)],
                            out_hbm.at[c, pl.ds(start, RPT)])

        @pl.when(s == NS - 1)
        def _():
            pltpu.sync_copy(acc.at[pl.ds(start, RPT_LAST)],
                            out_hbm.at[c, pl.ds(start, RPT_LAST)])

    return scatter_kernel


_SCAT64 = _sc_scatter(64)
_SCAT128 = _sc_scatter(128)


def _leaky(x):
    return jnp.where(x >= 0, x, 0.01 * x)


_BM = 1000
_BC = 2000


def _mm1_body(x_ref, w_ref, *y_refs):
    x = x_ref[...]
    for k in range(4):
        y_refs[k][...] = jnp.dot(x, w_ref[k],
                                 preferred_element_type=jnp.float32)


_MM1 = pl.pallas_call(
    _mm1_body,
    grid=(N // _BM,),
    in_specs=[pl.BlockSpec((_BM, 128), lambda i: (i, 0)),
              pl.BlockSpec((4, 128, 64), lambda i: (0, 0, 0))],
    out_specs=[pl.BlockSpec((_BM, 64), lambda i: (i, 0))] * 4,
    out_shape=[jax.ShapeDtypeStruct((N, 64), jnp.float32)] * 4,
)


def _make_combine(F, n_extra, bias, act):
    def body(*refs):
        refs = list(refs)
        o_ref = refs.pop()
        b_ref = refs.pop() if bias else None
        p_ref = refs.pop(0)
        t = p_ref[0] + p_ref[1]
        for r in refs:
            t = t + r[...]
        if b_ref is not None:
            t = t + b_ref[...]
        if act:
            t = _leaky(t)
        o_ref[...] = t

    in_specs = [pl.BlockSpec((2, _BC, F), lambda i: (0, i, 0))]
    in_specs += [pl.BlockSpec((_BC, F), lambda i: (i, 0))] * n_extra
    if bias:
        in_specs.append(pl.BlockSpec((1, F), lambda i: (0, 0)))
    return pl.pallas_call(
        body,
        grid=(N // _BC,),
        in_specs=in_specs,
        out_specs=pl.BlockSpec((_BC, F), lambda i: (i, 0)),
        out_shape=jax.ShapeDtypeStruct((N, F), jnp.float32),
    )


_COMB64_Y = _make_combine(64, 1, False, False)
_COMB64_YBA = _make_combine(64, 1, True, True)
_COMB64 = _make_combine(64, 0, False, False)
_COMB128 = _make_combine(128, 0, False, False)


def _make_mm4(fin, fout, act):
    def body(h0, h1, h2, h3, w_ref, b_ref, o_ref):
        acc = jnp.dot(h0[...], w_ref[0], preferred_element_type=jnp.float32)
        for k, h in enumerate((h1, h2, h3), start=1):
            acc = acc + jnp.dot(h[...], w_ref[k],
                                preferred_element_type=jnp.float32)
        acc = acc + b_ref[...]
        if act:
            acc = _leaky(acc)
        o_ref[...] = acc

    return pl.pallas_call(
        body,
        grid=(N // _BM,),
        in_specs=[pl.BlockSpec((_BM, fin), lambda i: (i, 0))] * 4
        + [pl.BlockSpec((4, fin, fout), lambda i: (0, 0, 0)),
           pl.BlockSpec((1, fout), lambda i: (0, 0))],
        out_specs=pl.BlockSpec((_BM, fout), lambda i: (i, 0)),
        out_shape=jax.ShapeDtypeStruct((N, fout), jnp.float32),
    )


_MM4_L2 = _make_mm4(64, 128, True)
_MM4_L3 = _make_mm4(128, 256, False)

_BP = 1000


def _pool_body(b_ref, h_ref, o_ref):
    @pl.when(pl.program_id(0) == 0)
    def _init():
        o_ref[...] = jnp.full((G, 256), -jnp.inf, jnp.float32)

    bi = b_ref[0]
    h = h_ref[...]
    lo = jnp.min(bi)
    hi = jnp.max(bi)

    def gbody(g, carry):
        m = bi == g
        v = jnp.max(jnp.where(m, h, -jnp.inf), axis=0, keepdims=True)
        o_ref[pl.ds(g, 1), :] = jnp.maximum(o_ref[pl.ds(g, 1), :], v)
        return carry

    lax.fori_loop(lo, hi + 1, gbody, 0)


_POOL = pl.pallas_call(
    _pool_body,
    grid=(N // _BP,),
    in_specs=[pl.BlockSpec((1, _BP, 1), lambda i: (i, 0, 0)),
              pl.BlockSpec((_BP, 256), lambda i: (i, 0))],
    out_specs=pl.BlockSpec((G, 256), lambda i: (0, 0)),
    out_shape=jax.ShapeDtypeStruct((G, 256), jnp.float32),
)


def _mlp_body(g_ref, w1, b1, w2, b2, w3, b3, o_ref):
    g = g_ref[...]
    g = jnp.where(jnp.isfinite(g), g, 0.0)
    h = jnp.maximum(
        jnp.dot(g, w1[...], preferred_element_type=jnp.float32) + b1[...], 0.0)
    h = jnp.maximum(
        jnp.dot(h, w2[...], preferred_element_type=jnp.float32) + b2[...], 0.0)
    o_ref[...] = jnp.dot(h, w3[...], preferred_element_type=jnp.float32) + b3[...]


_MLP = pl.pallas_call(
    _mlp_body,
    out_shape=jax.ShapeDtypeStruct((G, 4), jnp.float32),
)


def kernel(x, edge_index, batch, W1, b1, W2, b2, W3, b3,
           Wc1, bc1, Wc2, bc2, Wc3, bc3):
    src = edge_index[0]
    dst = edge_index[1]
    z64 = jnp.zeros((RPT_LAST, 64), jnp.float32)
    z128 = jnp.zeros((RPT_LAST, 128), jnp.float32)

    y0, y1, y2, y3 = _MM1(x, W1)
    p = _SCAT64(y3, src, dst, z64)
    t = _COMB64_Y(p, y2)
    p = _SCAT64(t, src, dst, z64)
    t = _COMB64_Y(p, y1)
    p = _SCAT64(t, src, dst, z64)
    h1 = _COMB64_YBA(p, y0, b1.reshape(1, 64))

    p = _SCAT64(h1, src, dst, z64)
    h2 = _COMB64(p)
    p = _SCAT64(h2, src, dst, z64)
    h3 = _COMB64(p)
    p = _SCAT64(h3, src, dst, z64)
    h4 = _COMB64(p)
    z = _MM4_L2(h1, h2, h3, h4, W2, b2.reshape(1, 128))

    p = _SCAT128(z, src, dst, z128)
    g2 = _COMB128(p)
    p = _SCAT128(g2, src, dst, z128)
    g3 = _COMB128(p)
    p = _SCAT128(g3, src, dst, z128)
    g4 = _COMB128(p)
    hfin = _MM4_L3(z, g2, g3, g4, W3, b3.reshape(1, 256))

    gpool = _POOL(batch.reshape(N // _BP, _BP, 1), hfin)
    return _MLP(gpool, Wc1, bc1.reshape(1, 1024), Wc2, bc2.reshape(1, 512),
                Wc3, bc3.reshape(1, 4))
